# baseline (speedup 1.0000x reference)
import functools

import jax
import jax.numpy as jnp
from jax import lax
from jax.experimental import pallas as pl
from jax.experimental.pallas import tpu as pltpu
from jax.experimental.pallas import tpu_sc as plsc

L = 4
D = 1024
F = 4096
H = 16
V = 16000
S = 2048
HD = D // H
EPS = 1e-3
SCALE = 1.0 / (HD ** 0.5)

NC = 2
NS = 16
NW = NC * NS
BPW = S // NW

BSR = 256
NR = S // BSR
VB = 3200
NVB = V // VB


def _sc_gather_body(table_hbm, idx_hbm, out_hbm, idx_v, rows_v, sem):
    wid = lax.axis_index("s") * NC + lax.axis_index("c")
    base = wid * BPW
    pltpu.sync_copy(idx_hbm.at[pl.ds(base, BPW)], idx_v)
    pltpu.async_copy(table_hbm.at[idx_v], rows_v, sem).wait()
    pltpu.sync_copy(rows_v, out_hbm.at[pl.ds(base, BPW)])


def _sc_gather(table, idx):
    return pl.kernel(
        _sc_gather_body,
        out_type=jax.ShapeDtypeStruct((S, D), jnp.float32),
        mesh=plsc.VectorSubcoreMesh(core_axis_name="c", subcore_axis_name="s"),
        scratch_types=[
            pltpu.VMEM((BPW,), jnp.int32),
            pltpu.VMEM((BPW, D), jnp.float32),
            pltpu.SemaphoreType.DMA,
        ],
    )(table, idx)


def _add_body(a_ref, b_ref, o_ref):
    o_ref[...] = a_ref[...] + b_ref[...]


def _posadd(emb, pos):
    return pl.pallas_call(
        _add_body,
        out_shape=jax.ShapeDtypeStruct((S, D), jnp.float32),
    )(emb, pos)


def _qkv_body(x_ref, wq_ref, wk_ref, wv_ref, bq_ref, bk_ref, bv_ref,
              q_ref, k_ref, v_ref):
    x = x_ref[...]
    q_ref[...] = jnp.dot(x, wq_ref[...], preferred_element_type=jnp.float32) + bq_ref[...]
    k_ref[...] = jnp.dot(x, wk_ref[...], preferred_element_type=jnp.float32) + bk_ref[...]
    v_ref[...] = jnp.dot(x, wv_ref[...], preferred_element_type=jnp.float32) + bv_ref[...]


def _qkv(x, wq, wk, wv, bq, bk, bv):
    w_spec = pl.BlockSpec((D, D), lambda i: (0, 0))
    b_spec = pl.BlockSpec((1, D), lambda i: (0, 0))
    r_spec = pl.BlockSpec((BSR, D), lambda i: (i, 0))
    return pl.pallas_call(
        _qkv_body,
        grid=(NR,),
        in_specs=[r_spec, w_spec, w_spec, w_spec, b_spec, b_spec, b_spec],
        out_specs=[r_spec, r_spec, r_spec],
        out_shape=[jax.ShapeDtypeStruct((S, D), jnp.float32)] * 3,
    )(x, wq, wk, wv, bq, bk, bv)


def _attn_body(q_ref, k_ref, v_ref, o_ref):
    i = pl.program_id(1)
    q = q_ref[0] * SCALE
    s = lax.dot_general(q, k_ref[0], (((1,), (1,)), ((), ())),
                        preferred_element_type=jnp.float32)
    row = i * BSR + lax.broadcasted_iota(jnp.int32, (BSR, S), 0)
    col = lax.broadcasted_iota(jnp.int32, (BSR, S), 1)
    s = jnp.where(col <= row, s, jnp.float32(-1e9))
    m = jnp.max(s, axis=-1, keepdims=True)
    p = jnp.exp(s - m)
    p = p / jnp.sum(p, axis=-1, keepdims=True)
    o_ref[0] = jnp.dot(p, v_ref[0], preferred_element_type=jnp.float32)


def _attention(qh, kh, vh):
    qo_spec = pl.BlockSpec((1, BSR, HD), lambda h, i: (h, i, 0))
    kv_spec = pl.BlockSpec((1, S, HD), lambda h, i: (h, 0, 0))
    return pl.pallas_call(
        _attn_body,
        grid=(H, NR),
        in_specs=[qo_spec, kv_spec, kv_spec],
        out_specs=qo_spec,
        out_shape=jax.ShapeDtypeStruct((H, S, HD), jnp.float32),
    )(qh, kh, vh)


def _layernorm(t, g, b):
    mu = jnp.mean(t, axis=-1, keepdims=True)
    var = jnp.mean(jnp.square(t - mu), axis=-1, keepdims=True)
    return (t - mu) / jnp.sqrt(var + EPS) * g + b


def _oproj_body(o_ref, wo_ref, bo_ref, x_ref, g_ref, b_ref, y_ref):
    t = jnp.dot(o_ref[...], wo_ref[...], preferred_element_type=jnp.float32)
    t = t + bo_ref[...] + x_ref[...]
    y_ref[...] = _layernorm(t, g_ref[...], b_ref[...])


def _oproj_ln(o, wo, bo, x, g, b):
    w_spec = pl.BlockSpec((D, D), lambda i: (0, 0))
    b_spec = pl.BlockSpec((1, D), lambda i: (0, 0))
    r_spec = pl.BlockSpec((BSR, D), lambda i: (i, 0))
    return pl.pallas_call(
        _oproj_body,
        grid=(NR,),
        in_specs=[r_spec, w_spec, b_spec, r_spec, b_spec, b_spec],
        out_specs=r_spec,
        out_shape=jax.ShapeDtypeStruct((S, D), jnp.float32),
    )(o, wo, bo, x, g, b)


def _ffn_body(y_ref, w1_ref, b1_ref, w2_ref, b2_ref, g_ref, b_ref, out_ref):
    y = y_ref[...]
    h = jnp.dot(y, w1_ref[...], preferred_element_type=jnp.float32) + b1_ref[...]
    h = jnp.maximum(h, 0.0)
    t = jnp.dot(h, w2_ref[...], preferred_element_type=jnp.float32)
    t = t + b2_ref[...] + y
    out_ref[...] = _layernorm(t, g_ref[...], b_ref[...])


def _ffn_ln(y, w1, b1, w2, b2, g, b):
    r_spec = pl.BlockSpec((BSR, D), lambda i: (i, 0))
    bD_spec = pl.BlockSpec((1, D), lambda i: (0, 0))
    return pl.pallas_call(
        _ffn_body,
        grid=(NR,),
        in_specs=[
            r_spec,
            pl.BlockSpec((D, F), lambda i: (0, 0)),
            pl.BlockSpec((1, F), lambda i: (0, 0)),
            pl.BlockSpec((F, D), lambda i: (0, 0)),
            bD_spec, bD_spec, bD_spec,
        ],
        out_specs=r_spec,
        out_shape=jax.ShapeDtypeStruct((S, D), jnp.float32),
    )(y, w1, b1, w2, b2, g, b)


def _out_body(x_ref, w_ref, b_ref, o_ref):
    o_ref[...] = (jnp.dot(x_ref[...], w_ref[...], preferred_element_type=jnp.float32)
                  + b_ref[...])


def _outproj(x, wout, bout):
    return pl.pallas_call(
        _out_body,
        grid=(NVB, NR),
        in_specs=[
            pl.BlockSpec((BSR, D), lambda j, i: (i, 0)),
            pl.BlockSpec((D, VB), lambda j, i: (0, j)),
            pl.BlockSpec((1, VB), lambda j, i: (0, j)),
        ],
        out_specs=pl.BlockSpec((BSR, VB), lambda j, i: (i, j)),
        out_shape=jax.ShapeDtypeStruct((S, V), jnp.float32),
    )(x, wout, bout)


def _tc_forward(x, p):
    for l in range(L):
        q, k, v = _qkv(x, p['Wq'][l], p['Wk'][l], p['Wv'][l],
                       p['bq'][l][None, :], p['bk'][l][None, :], p['bv'][l][None, :])
        qh = q.reshape(S, H, HD).transpose(1, 0, 2)
        kh = k.reshape(S, H, HD).transpose(1, 0, 2)
        vh = v.reshape(S, H, HD).transpose(1, 0, 2)
        oh = _attention(qh, kh, vh)
        o = oh.transpose(1, 0, 2).reshape(S, D)
        y = _oproj_ln(o, p['Wo'][l], p['bo'][l][None, :], x,
                      p['ln1_g'][l][None, :], p['ln1_b'][l][None, :])
        x = _ffn_ln(y, p['W1'][l], p['b1'][l][None, :], p['W2'][l], p['b2'][l][None, :],
                    p['ln2_g'][l][None, :], p['ln2_b'][l][None, :])
    logits = _outproj(x, p['Wout'], p['bout'][None, :])
    return logits, x


def kernel(inputs, params):
    b, s = inputs.shape
    idx = inputs.reshape(-1)
    emb = _sc_gather(params['tok_emb'], idx)
    x = _posadd(emb, params['pos_emb'])
    logits, x = _tc_forward(x, params)
    return logits[None, :, :], x[None, :, :]


# speedup vs baseline: 1.1777x; 1.1777x over previous
import functools

import jax
import jax.numpy as jnp
from jax import lax
from jax.experimental import pallas as pl
from jax.experimental.pallas import tpu as pltpu
from jax.experimental.pallas import tpu_sc as plsc

L = 4
D = 1024
F = 4096
H = 16
V = 16000
S = 2048
HD = D // H
EPS = 1e-3
SCALE = 1.0 / (HD ** 0.5)

NC = 2
NS = 16
NW = NC * NS
BPW = S // NW

BSR = 256
NR = S // BSR
VB = 3200
NVB = V // VB


def _sc_gather_body(table_hbm, idx_hbm, out_hbm, idx_v, rows_v, sem):
    wid = lax.axis_index("s") * NC + lax.axis_index("c")
    base = wid * BPW
    pltpu.sync_copy(idx_hbm.at[pl.ds(base, BPW)], idx_v)
    pltpu.async_copy(table_hbm.at[idx_v], rows_v, sem).wait()
    pltpu.sync_copy(rows_v, out_hbm.at[pl.ds(base, BPW)])


def _sc_gather(table, idx):
    return pl.kernel(
        _sc_gather_body,
        out_type=jax.ShapeDtypeStruct((S, D), jnp.float32),
        mesh=plsc.VectorSubcoreMesh(core_axis_name="c", subcore_axis_name="s"),
        scratch_types=[
            pltpu.VMEM((BPW,), jnp.int32),
            pltpu.VMEM((BPW, D), jnp.float32),
            pltpu.SemaphoreType.DMA,
        ],
    )(table, idx)


def _add_body(a_ref, b_ref, o_ref):
    o_ref[...] = a_ref[...] + b_ref[...]


def _posadd(emb, pos):
    return pl.pallas_call(
        _add_body,
        out_shape=jax.ShapeDtypeStruct((S, D), jnp.float32),
    )(emb, pos)


def _qkv_body(x_ref, wq_ref, wk_ref, wv_ref, bq_ref, bk_ref, bv_ref,
              q_ref, k_ref, v_ref):
    x = x_ref[...].astype(jnp.bfloat16)
    q = jnp.dot(x, wq_ref[...], preferred_element_type=jnp.float32) + bq_ref[...]
    k = jnp.dot(x, wk_ref[...], preferred_element_type=jnp.float32) + bk_ref[...]
    v = jnp.dot(x, wv_ref[...], preferred_element_type=jnp.float32) + bv_ref[...]
    q_ref[...] = q.astype(jnp.bfloat16)
    k_ref[...] = k.astype(jnp.bfloat16)
    v_ref[...] = v.astype(jnp.bfloat16)


def _qkv(x, wq, wk, wv, bq, bk, bv):
    w_spec = pl.BlockSpec((D, D), lambda i: (0, 0))
    b_spec = pl.BlockSpec((1, D), lambda i: (0, 0))
    r_spec = pl.BlockSpec((BSR, D), lambda i: (i, 0))
    return pl.pallas_call(
        _qkv_body,
        grid=(NR,),
        in_specs=[r_spec, w_spec, w_spec, w_spec, b_spec, b_spec, b_spec],
        out_specs=[r_spec, r_spec, r_spec],
        out_shape=[jax.ShapeDtypeStruct((S, D), jnp.bfloat16)] * 3,
    )(x, wq, wk, wv, bq, bk, bv)


def _attn_body(q_ref, k_ref, v_ref, o_ref):
    i = pl.program_id(1)
    s = lax.dot_general(q_ref[0], k_ref[0], (((1,), (1,)), ((), ())),
                        preferred_element_type=jnp.float32) * SCALE
    row = i * BSR + lax.broadcasted_iota(jnp.int32, (BSR, S), 0)
    col = lax.broadcasted_iota(jnp.int32, (BSR, S), 1)
    s = jnp.where(col <= row, s, jnp.float32(-1e9))
    m = jnp.max(s, axis=-1, keepdims=True)
    p = jnp.exp(s - m)
    p = p / jnp.sum(p, axis=-1, keepdims=True)
    o_ref[0] = jnp.dot(p.astype(jnp.bfloat16), v_ref[0],
                       preferred_element_type=jnp.float32).astype(jnp.bfloat16)


def _attention(qh, kh, vh):
    qo_spec = pl.BlockSpec((1, BSR, HD), lambda h, i: (h, i, 0))
    kv_spec = pl.BlockSpec((1, S, HD), lambda h, i: (h, 0, 0))
    return pl.pallas_call(
        _attn_body,
        grid=(H, NR),
        in_specs=[qo_spec, kv_spec, kv_spec],
        out_specs=qo_spec,
        out_shape=jax.ShapeDtypeStruct((H, S, HD), jnp.bfloat16),
    )(qh, kh, vh)


def _layernorm(t, g, b):
    mu = jnp.mean(t, axis=-1, keepdims=True)
    var = jnp.mean(jnp.square(t - mu), axis=-1, keepdims=True)
    return (t - mu) / jnp.sqrt(var + EPS) * g + b


def _oproj_body(o_ref, wo_ref, bo_ref, x_ref, g_ref, b_ref, y_ref):
    t = jnp.dot(o_ref[...], wo_ref[...], preferred_element_type=jnp.float32)

    t = t + bo_ref[...] + x_ref[...]
    y_ref[...] = _layernorm(t, g_ref[...], b_ref[...])


def _oproj_ln(o, wo, bo, x, g, b):
    w_spec = pl.BlockSpec((D, D), lambda i: (0, 0))
    b_spec = pl.BlockSpec((1, D), lambda i: (0, 0))
    r_spec = pl.BlockSpec((BSR, D), lambda i: (i, 0))
    return pl.pallas_call(
        _oproj_body,
        grid=(NR,),
        in_specs=[r_spec, w_spec, b_spec, r_spec, b_spec, b_spec],
        out_specs=r_spec,
        out_shape=jax.ShapeDtypeStruct((S, D), jnp.float32),
    )(o, wo, bo, x, g, b)


def _ffn_body(y_ref, w1_ref, b1_ref, w2_ref, b2_ref, g_ref, b_ref, out_ref):
    y = y_ref[...]
    h = jnp.dot(y.astype(jnp.bfloat16), w1_ref[...],
                preferred_element_type=jnp.float32) + b1_ref[...]
    h = jnp.maximum(h, 0.0)
    t = jnp.dot(h.astype(jnp.bfloat16), w2_ref[...],
                preferred_element_type=jnp.float32)
    t = t + b2_ref[...] + y
    out_ref[...] = _layernorm(t, g_ref[...], b_ref[...])


def _ffn_ln(y, w1, b1, w2, b2, g, b):
    r_spec = pl.BlockSpec((BSR, D), lambda i: (i, 0))
    bD_spec = pl.BlockSpec((1, D), lambda i: (0, 0))
    return pl.pallas_call(
        _ffn_body,
        grid=(NR,),
        in_specs=[
            r_spec,
            pl.BlockSpec((D, F), lambda i: (0, 0)),
            pl.BlockSpec((1, F), lambda i: (0, 0)),
            pl.BlockSpec((F, D), lambda i: (0, 0)),
            bD_spec, bD_spec, bD_spec,
        ],
        out_specs=r_spec,
        out_shape=jax.ShapeDtypeStruct((S, D), jnp.float32),
    )(y, w1, b1, w2, b2, g, b)


def _out_body(x_ref, w_ref, b_ref, o_ref):
    o_ref[...] = (jnp.dot(x_ref[...].astype(jnp.bfloat16), w_ref[...],
                          preferred_element_type=jnp.float32) + b_ref[...])


def _outproj(x, wout, bout):
    return pl.pallas_call(
        _out_body,
        grid=(NVB, NR),
        in_specs=[
            pl.BlockSpec((BSR, D), lambda j, i: (i, 0)),
            pl.BlockSpec((D, VB), lambda j, i: (0, j)),
            pl.BlockSpec((1, VB), lambda j, i: (0, j)),
        ],
        out_specs=pl.BlockSpec((BSR, VB), lambda j, i: (i, j)),
        out_shape=jax.ShapeDtypeStruct((S, V), jnp.float32),
    )(x, wout, bout)


def _tc_forward(x, p):
    bf = jnp.bfloat16
    for l in range(L):
        q, k, v = _qkv(x, p['Wq'][l].astype(bf), p['Wk'][l].astype(bf), p['Wv'][l].astype(bf),
                       p['bq'][l][None, :], p['bk'][l][None, :], p['bv'][l][None, :])
        qh = q.reshape(S, H, HD).transpose(1, 0, 2)
        kh = k.reshape(S, H, HD).transpose(1, 0, 2)
        vh = v.reshape(S, H, HD).transpose(1, 0, 2)
        oh = _attention(qh, kh, vh)
        o = oh.transpose(1, 0, 2).reshape(S, D)
        y = _oproj_ln(o, p['Wo'][l].astype(bf), p['bo'][l][None, :], x,
                      p['ln1_g'][l][None, :], p['ln1_b'][l][None, :])
        x = _ffn_ln(y, p['W1'][l].astype(bf), p['b1'][l][None, :],
                    p['W2'][l].astype(bf), p['b2'][l][None, :],
                    p['ln2_g'][l][None, :], p['ln2_b'][l][None, :])
    logits = _outproj(x, p['Wout'].astype(bf), p['bout'][None, :])
    return logits, x


def kernel(inputs, params):
    b, s = inputs.shape
    idx = inputs.reshape(-1)
    emb = _sc_gather(params['tok_emb'], idx)
    x = _posadd(emb, params['pos_emb'])
    logits, x = _tc_forward(x, params)
    return logits[None, :, :], x[None, :, :]
